# Initial kernel scaffold; baseline (speedup 1.0000x reference)
import jax
import jax.numpy as jnp
from jax.experimental import pallas as pl


def kernel(x, w1p, s1p, b1p, w2f, s2t, b2t, w3b, s3t, b3t):
    raise NotImplementedError("write your pallas kernel here")


# speedup vs baseline: 1.0095x; 1.0095x over previous
import functools

import numpy as np
import jax
import jax.numpy as jnp
from jax.experimental import pallas as pl
from jax.experimental.pallas import tpu as pltpu


def _band_w(width):
    b = np.zeros((3, width, width), np.float32)
    for k in range(3):
        for w in range(width):
            v = w - k + 1
            if 0 <= v < width:
                b[k, w, v] = 1.0
    return b


def _bottleneck_body(x_ref, w1_ref, s1_ref, b1_ref, w2_ref, s2_ref, b2_ref,
                     w3_ref, s3_ref, b3_ref, o_ref, hbuf_ref, lhs_ref, *,
                     d_size, h_size):
    m = x_ref.shape[0]
    cp = hbuf_ref.shape[1]
    cdt = lhs_ref.dtype

    x = x_ref[...]

    h1 = jnp.dot(x.astype(cdt), w1_ref[...], preferred_element_type=jnp.float32)
    h1 = jnp.maximum(h1 * s1_ref[...] + b1_ref[...], 0.0)

    zeros = jnp.zeros((16, cp), jnp.float32)
    hbuf_ref[0:16, :] = zeros
    hbuf_ref[m + 16:m + 32, :] = zeros
    hbuf_ref[16:m + 16, :] = h1

    r = jax.lax.broadcasted_iota(jnp.int32, (m, 1), 0)
    h_idx = r % h_size
    d_idx = (r // h_size) % d_size

    for kd in range(3):
        for kh in range(3):
            t = kd * 3 + kh
            if kd == 1 and kh == 1:
                lhs_ref[:, t * cp:(t + 1) * cp] = h1.astype(cdt)
                continue
            off = 16 + (kd - 1) * h_size + (kh - 1)
            conds = []
            if kd == 0:
                conds.append(d_idx > 0)
            elif kd == 2:
                conds.append(d_idx < d_size - 1)
            if kh == 0:
                conds.append(h_idx > 0)
            elif kh == 2:
                conds.append(h_idx < h_size - 1)
            mask = conds[0]
            for c in conds[1:]:
                mask = jnp.logical_and(mask, c)
            src = hbuf_ref[off:off + m, :]
            lhs_ref[:, t * cp:(t + 1) * cp] = jnp.where(mask, src, 0.0).astype(cdt)

    h2 = jnp.dot(lhs_ref[...], w2_ref[...], preferred_element_type=jnp.float32)
    h2 = jnp.maximum(h2 * s2_ref[...] + b2_ref[...], 0.0)

    h3 = jnp.dot(h2.astype(cdt), w3_ref[...], preferred_element_type=jnp.float32)
    h3 = h3 * s3_ref[...] + b3_ref[...]
    o_ref[...] = jnp.maximum(h3 + x, 0.0).astype(o_ref.dtype)


def kernel(x, w1p, s1p, b1p, w2f, s2t, b2t, w3b, s3t, b3t):
    N, Cin, D, H, W = x.shape
    P = w2f.shape[1] // (H * W)
    Wp = W + 2
    rowp = w1p.shape[1]
    cdt = w1p.dtype

    base = (Wp + 1) * P
    w1e = w1p[:Cin, base:base + P]
    s1e = s1p[0, base:base + P]
    b1e = b1p[0, base:base + P]

    taps = np.array([kh * Wp + kw for kh in range(3) for kw in range(3)])
    w2r = w2f[:, :P].reshape(3, rowp // P, P, P)
    w2small = w2r[:, taps].reshape(3, 3, 3, P, P)
    s2e = s2t[0, :P]
    b2e = b2t[0, :P]

    w3e = w3b[:P, :Cin]
    s3e = s3t[0, :Cin]
    b3e = b3t[0, :Cin]

    eye_w = jnp.eye(W, dtype=cdt)
    w1k = jnp.kron(eye_w, w1e)
    w3k = jnp.kron(eye_w, w3e)
    band = jnp.asarray(_band_w(W), dtype=cdt)
    w2k = jnp.einsum('dhkcp,kwv->dhwcvp', w2small, band)
    w2k = w2k.reshape(9 * W * P, W * P)

    s1t_ = jnp.tile(s1e, W)[None, :]
    b1t_ = jnp.tile(b1e, W)[None, :]
    s2t_ = jnp.tile(s2e, W)[None, :]
    b2t_ = jnp.tile(b2e, W)[None, :]
    s3t_ = jnp.tile(s3e, W)[None, :]
    b3t_ = jnp.tile(b3e, W)[None, :]

    x2d = jnp.transpose(x, (0, 2, 3, 4, 1)).reshape(N * D * H, W * Cin)

    b_blk = 8
    while N % b_blk:
        b_blk //= 2
    m = b_blk * D * H
    grid = (N // b_blk,)
    kin = W * Cin
    kmid = W * P
    ops = (w1k, s1t_, b1t_, w2k, s2t_, b2t_, w3k, s3t_, b3t_)

    weight_specs = [pl.BlockSpec(a.shape, lambda g: (0,) * a.ndim) for a in ops]
    in_specs = [pl.BlockSpec((m, kin), lambda g: (g, 0))] + weight_specs
    out_specs = pl.BlockSpec((m, kin), lambda g: (g, 0))

    body = functools.partial(_bottleneck_body, d_size=D, h_size=H)
    y2d = pl.pallas_call(
        body,
        out_shape=jax.ShapeDtypeStruct((N * D * H, kin), x.dtype),
        grid_spec=pltpu.PrefetchScalarGridSpec(
            num_scalar_prefetch=0,
            grid=grid,
            in_specs=in_specs,
            out_specs=out_specs,
            scratch_shapes=[
                pltpu.VMEM((m + 32, kmid), jnp.float32),
                pltpu.VMEM((m, 9 * kmid), cdt),
            ]),
        compiler_params=pltpu.CompilerParams(
            dimension_semantics=("parallel",),
            vmem_limit_bytes=64 << 20),
    )(x2d, *ops)

    y = y2d.reshape(N, D, H, W, Cin)
    return jnp.transpose(y, (0, 4, 1, 2, 3))


# speedup vs baseline: 1.1841x; 1.1729x over previous
import functools

import numpy as np
import jax
import jax.numpy as jnp
from jax.experimental import pallas as pl
from jax.experimental.pallas import tpu as pltpu


def _bottleneck_body(x_ref, w1_ref, s1_ref, b1_ref, w2k0_ref, w2k1_ref,
                     w2k2_ref, s2_ref, b2_ref, w3_ref, s3_ref, b3_ref,
                     o_ref, hpad_ref, r2_ref, *, d_size, h_size, w_size,
                     b_blk, planes):
    s_size = x_ref.shape[1]
    mh = hpad_ref.shape[0]
    cdt = r2_ref.dtype

    x = x_ref[...]

    h1 = jnp.dot(w1_ref[...], x.astype(cdt),
                 preferred_element_type=jnp.float32)
    h1 = jnp.maximum(h1 * s1_ref[...] + b1_ref[...], 0.0)

    hpad_ref[:, 0:128] = jnp.zeros((mh, 128), cdt)
    hpad_ref[:, s_size + 128:s_size + 256] = jnp.zeros((mh, 128), cdt)
    hpad_ref[:, 128:s_size + 128] = h1.astype(cdt)

    lane = jax.lax.broadcasted_iota(jnp.int32, (1, s_size), 1)
    h_of_lane = (lane // w_size) % h_size
    w_of_lane = lane % w_size

    for kd in range(3):
        for kh in range(3):
            t = kd * 3 + kh
            off = 128 + (kd - 1) * h_size * w_size + (kh - 1) * w_size
            src = hpad_ref[:, off:off + s_size]
            if kh == 0:
                src = jnp.where(h_of_lane != 0, src, 0)
            elif kh == 2:
                src = jnp.where(h_of_lane != h_size - 1, src, 0)
            r2_ref[t * mh:(t + 1) * mh, :] = src

    r2 = r2_ref[...]
    y0 = jnp.dot(w2k0_ref[...], r2, preferred_element_type=jnp.float32)
    y1 = jnp.dot(w2k1_ref[...], r2, preferred_element_type=jnp.float32)
    y2 = jnp.dot(w2k2_ref[...], r2, preferred_element_type=jnp.float32)

    h2 = y1
    h2 = h2 + jnp.where(w_of_lane != 0, jnp.roll(y0, 1, axis=1), 0.0)
    h2 = h2 + jnp.where(w_of_lane != w_size - 1, jnp.roll(y2, -1, axis=1), 0.0)
    h2 = jnp.maximum(h2 * s2_ref[...] + b2_ref[...], 0.0)

    h3 = jnp.dot(w3_ref[...], h2.astype(cdt),
                 preferred_element_type=jnp.float32)
    h3 = h3 * s3_ref[...] + b3_ref[...]
    o_ref[...] = jnp.maximum(h3 + x, 0.0).astype(o_ref.dtype)


def kernel(x, w1p, s1p, b1p, w2f, s2t, b2t, w3b, s3t, b3t):
    N, Cin, D, H, W = x.shape
    S = D * H * W
    P = w2f.shape[1] // (H * W)
    Wp = W + 2
    rowp = w1p.shape[1]
    cdt = w1p.dtype

    base = (Wp + 1) * P
    w1e = w1p[:Cin, base:base + P]
    s1e = s1p[0, base:base + P]
    b1e = b1p[0, base:base + P]

    taps = np.array([kh * Wp + kw for kh in range(3) for kw in range(3)])
    w2r = w2f[:, :P].reshape(3, rowp // P, P, P)
    w2small = w2r[:, taps].reshape(3, 3, 3, P, P)
    s2e = s2t[0, :P]
    b2e = b2t[0, :P]

    w3e = w3b[:P, :Cin]
    s3e = s3t[0, :Cin]
    b3e = b3t[0, :Cin]

    b_blk = 16
    while N % b_blk:
        b_blk //= 2
    eye_b = jnp.eye(b_blk, dtype=cdt)
    w1bd = jnp.kron(eye_b, w1e.T)
    w3bd = jnp.kron(eye_b, w3e.T)
    w2bd = [jnp.concatenate(
        [jnp.kron(eye_b, w2small[kd, kh, kw].T)
         for kd in range(3) for kh in range(3)], axis=1)
        for kw in range(3)]

    s1c = jnp.tile(s1e, b_blk)[:, None]
    b1c = jnp.tile(b1e, b_blk)[:, None]
    s2c = jnp.tile(s2e, b_blk)[:, None]
    b2c = jnp.tile(b2e, b_blk)[:, None]
    s3c = jnp.tile(s3e, b_blk)[:, None]
    b3c = jnp.tile(b3e, b_blk)[:, None]

    x2d = x.reshape(N * Cin, S)
    mx = b_blk * Cin
    mh = b_blk * P
    grid = (N // b_blk,)

    ops = (w1bd, s1c, b1c, w2bd[0], w2bd[1], w2bd[2], s2c, b2c,
           w3bd, s3c, b3c)
    weight_specs = [pl.BlockSpec(a.shape, lambda g: (0, 0)) for a in ops]
    in_specs = [pl.BlockSpec((mx, S), lambda g: (g, 0))] + weight_specs
    out_specs = pl.BlockSpec((mx, S), lambda g: (g, 0))

    body = functools.partial(_bottleneck_body, d_size=D, h_size=H, w_size=W,
                             b_blk=b_blk, planes=P)
    y2d = pl.pallas_call(
        body,
        out_shape=jax.ShapeDtypeStruct((N * Cin, S), x.dtype),
        grid_spec=pltpu.PrefetchScalarGridSpec(
            num_scalar_prefetch=0,
            grid=grid,
            in_specs=in_specs,
            out_specs=out_specs,
            scratch_shapes=[
                pltpu.VMEM((mh, S + 256), cdt),
                pltpu.VMEM((9 * mh, S), cdt),
            ]),
        compiler_params=pltpu.CompilerParams(
            dimension_semantics=("parallel",),
            vmem_limit_bytes=64 << 20),
    )(x2d, *ops)

    return y2d.reshape(N, Cin, D, H, W)


# speedup vs baseline: 1.2846x; 1.0849x over previous
import functools

import numpy as np
import jax
import jax.numpy as jnp
from jax.experimental import pallas as pl
from jax.experimental.pallas import tpu as pltpu


def _bottleneck_body(x_ref, w1_ref, w2_ref, w3_ref, sb12_ref, sb3_ref,
                     o_ref, hpad_ref, r2_ref, *, d_size, h_size, w_size,
                     b_blk, planes):
    s_size = x_ref.shape[1]
    mh = hpad_ref.shape[0]
    cdt = r2_ref.dtype

    x = x_ref[...]

    h1 = jnp.dot(w1_ref[...], x.astype(cdt),
                 preferred_element_type=jnp.float32)
    h1 = jnp.maximum(h1 * sb12_ref[:, 0:1] + sb12_ref[:, 1:2], 0.0)

    hpad_ref[:, 0:128] = jnp.zeros((mh, 128), cdt)
    hpad_ref[:, s_size + 128:s_size + 256] = jnp.zeros((mh, 128), cdt)
    hpad_ref[:, 128:s_size + 128] = h1.astype(cdt)

    lane = jax.lax.broadcasted_iota(jnp.int32, (1, s_size), 1)
    h_of_lane = (lane // w_size) % h_size
    w_of_lane = lane % w_size

    for kd in range(3):
        for kh in range(3):
            t = kd * 3 + kh
            off = 128 + (kd - 1) * h_size * w_size + (kh - 1) * w_size
            src = hpad_ref[:, off:off + s_size]
            if kh == 0:
                src = jnp.where(h_of_lane != 0, src, 0)
            elif kh == 2:
                src = jnp.where(h_of_lane != h_size - 1, src, 0)
            r2_ref[t * mh:(t + 1) * mh, :] = src

    r2 = r2_ref[...]
    y0 = jnp.dot(w2_ref[0], r2, preferred_element_type=jnp.float32)
    y1 = jnp.dot(w2_ref[1], r2, preferred_element_type=jnp.float32)
    y2 = jnp.dot(w2_ref[2], r2, preferred_element_type=jnp.float32)

    h2 = y1
    h2 = h2 + jnp.where(w_of_lane != 0, jnp.roll(y0, 1, axis=1), 0.0)
    h2 = h2 + jnp.where(w_of_lane != w_size - 1, jnp.roll(y2, -1, axis=1), 0.0)
    h2 = jnp.maximum(h2 * sb12_ref[:, 2:3] + sb12_ref[:, 3:4], 0.0)

    h3 = jnp.dot(w3_ref[...], h2.astype(cdt),
                 preferred_element_type=jnp.float32)
    h3 = h3 * sb3_ref[:, 0:1] + sb3_ref[:, 1:2]
    o_ref[...] = jnp.maximum(h3 + x, 0.0).astype(o_ref.dtype)


def kernel(x, w1p, s1p, b1p, w2f, s2t, b2t, w3b, s3t, b3t):
    N, Cin, D, H, W = x.shape
    S = D * H * W
    P = w2f.shape[1] // (H * W)
    Wp = W + 2
    rowp = w1p.shape[1]
    cdt = w1p.dtype

    base = (Wp + 1) * P
    w1e = w1p[:Cin, base:base + P]
    s1e = s1p[0, base:base + P]
    b1e = b1p[0, base:base + P]

    taps = np.array([kh * Wp + kw for kh in range(3) for kw in range(3)])
    w2r = w2f[:, :P].reshape(3, rowp // P, P, P)
    w2small = w2r[:, taps].reshape(3, 3, 3, P, P)
    s2e = s2t[0, :P]
    b2e = b2t[0, :P]

    w3e = w3b[:P, :Cin]
    s3e = s3t[0, :Cin]
    b3e = b3t[0, :Cin]

    b_blk = 16
    while N % b_blk:
        b_blk //= 2
    eye_b = np.eye(b_blk, dtype=np.float32)
    w1bd = jnp.einsum('qr,cp->qprc', eye_b, w1e).reshape(
        b_blk * P, b_blk * Cin).astype(cdt)
    w3bd = jnp.einsum('qr,pc->qcrp', eye_b, w3e).reshape(
        b_blk * Cin, b_blk * P).astype(cdt)
    w2bd = jnp.einsum('dhkcp,qr->kqpdhrc', w2small, eye_b).reshape(
        3, b_blk * P, 9 * b_blk * P).astype(cdt)

    sb12 = jnp.stack([jnp.tile(s1e, b_blk), jnp.tile(b1e, b_blk),
                      jnp.tile(s2e, b_blk), jnp.tile(b2e, b_blk)], axis=1)
    sb3 = jnp.stack([jnp.tile(s3e, b_blk), jnp.tile(b3e, b_blk)], axis=1)

    x2d = x.reshape(N * Cin, S)
    mx = b_blk * Cin
    mh = b_blk * P
    grid = (N // b_blk,)

    ops = (w1bd, w2bd, w3bd, sb12, sb3)
    weight_specs = [pl.BlockSpec(a.shape, lambda g, nd=a.ndim: (0,) * nd)
                    for a in ops]
    in_specs = [pl.BlockSpec((mx, S), lambda g: (g, 0))] + weight_specs
    out_specs = pl.BlockSpec((mx, S), lambda g: (g, 0))

    body = functools.partial(_bottleneck_body, d_size=D, h_size=H, w_size=W,
                             b_blk=b_blk, planes=P)
    y2d = pl.pallas_call(
        body,
        out_shape=jax.ShapeDtypeStruct((N * Cin, S), x.dtype),
        grid_spec=pltpu.PrefetchScalarGridSpec(
            num_scalar_prefetch=0,
            grid=grid,
            in_specs=in_specs,
            out_specs=out_specs,
            scratch_shapes=[
                pltpu.VMEM((mh, S + 256), cdt),
                pltpu.VMEM((9 * mh, S), cdt),
            ]),
        compiler_params=pltpu.CompilerParams(
            dimension_semantics=("parallel",),
            vmem_limit_bytes=64 << 20),
    )(x2d, *ops)

    return y2d.reshape(N, Cin, D, H, W)


# speedup vs baseline: 1.3133x; 1.0223x over previous
import functools

import numpy as np
import jax
import jax.numpy as jnp
from jax.experimental import pallas as pl
from jax.experimental.pallas import tpu as pltpu


def _bottleneck_body(x_ref, w1_ref, w2_ref, w3_ref, sb12_ref, sb3_ref,
                     o_ref, hpad_ref, r2_ref, *, d_size, h_size, w_size,
                     b_blk, planes):
    s_size = x_ref.shape[1]
    mh = hpad_ref.shape[0]
    cdt = r2_ref.dtype

    x = x_ref[...]

    h1 = jnp.dot(w1_ref[...], x.astype(cdt),
                 preferred_element_type=jnp.float32)
    h1 = jnp.maximum(h1 * sb12_ref[:, 0:1] + sb12_ref[:, 1:2], 0.0)

    hpad_ref[:, 0:128] = jnp.zeros((mh, 128), cdt)
    hpad_ref[:, s_size + 128:s_size + 256] = jnp.zeros((mh, 128), cdt)
    hpad_ref[:, 128:s_size + 128] = h1.astype(cdt)

    lane = jax.lax.broadcasted_iota(jnp.int32, (1, s_size), 1)
    h_of_lane = (lane // w_size) % h_size
    w_of_lane = lane % w_size

    for kd in range(3):
        for kh in range(3):
            t = kd * 3 + kh
            off = 128 + (kd - 1) * h_size * w_size + (kh - 1) * w_size
            src = hpad_ref[:, off:off + s_size]
            if kh == 0:
                src = jnp.where(h_of_lane != 0, src, 0)
            elif kh == 2:
                src = jnp.where(h_of_lane != h_size - 1, src, 0)
            r2_ref[t * mh:(t + 1) * mh, :] = src

    r2 = r2_ref[...]
    y0 = jnp.dot(w2_ref[0], r2, preferred_element_type=jnp.float32)
    y1 = jnp.dot(w2_ref[1], r2, preferred_element_type=jnp.float32)
    y2 = jnp.dot(w2_ref[2], r2, preferred_element_type=jnp.float32)

    h2 = y1
    h2 = h2 + jnp.where(w_of_lane != 0, jnp.roll(y0, 1, axis=1), 0.0)
    h2 = h2 + jnp.where(w_of_lane != w_size - 1, jnp.roll(y2, -1, axis=1), 0.0)
    h2 = jnp.maximum(h2 * sb12_ref[:, 2:3] + sb12_ref[:, 3:4], 0.0)

    h3 = jnp.dot(w3_ref[...], h2.astype(cdt),
                 preferred_element_type=jnp.float32)
    h3 = h3 * sb3_ref[:, 0:1] + sb3_ref[:, 1:2]
    o_ref[...] = jnp.maximum(h3 + x, 0.0).astype(o_ref.dtype)


def kernel(x, w1p, s1p, b1p, w2f, s2t, b2t, w3b, s3t, b3t):
    N, Cin, D, H, W = x.shape
    S = D * H * W
    P = w2f.shape[1] // (H * W)
    Wp = W + 2
    rowp = w1p.shape[1]
    cdt = w1p.dtype

    base = (Wp + 1) * P
    w1e = w1p[:Cin, base:base + P]
    s1e = s1p[0, base:base + P]
    b1e = b1p[0, base:base + P]

    taps = np.array([kh * Wp + kw for kh in range(3) for kw in range(3)])
    w2r = w2f[:, :P].reshape(3, rowp // P, P, P)
    w2small = w2r[:, taps].reshape(3, 3, 3, P, P)
    s2e = s2t[0, :P]
    b2e = b2t[0, :P]

    w3e = w3b[:P, :Cin]
    s3e = s3t[0, :Cin]
    b3e = b3t[0, :Cin]

    b_blk = 16
    while N % b_blk:
        b_blk //= 2

    def _bd_mask(br, bc, per):
        i = np.arange(b_blk * br)[:, None] // br
        j = np.arange(per * b_blk * bc)[None, :] % (b_blk * bc) // bc
        return (i == j).astype(np.float32)

    w1bd = (jnp.tile(w1e.T, (b_blk, b_blk))
            * _bd_mask(P, Cin, 1)).astype(cdt)
    w3bd = (jnp.tile(w3e.T, (b_blk, b_blk))
            * _bd_mask(Cin, P, 1)).astype(cdt)
    w2c = jnp.transpose(w2small, (2, 4, 0, 1, 3)).reshape(3, P, 9, P)
    w2row = jnp.broadcast_to(w2c[:, :, :, None, :],
                             (3, P, 9, b_blk, P)).reshape(3, P, 9 * b_blk * P)
    w2bd = (jnp.tile(w2row, (1, b_blk, 1))
            * _bd_mask(P, P, 9)[None]).astype(cdt)

    sb12 = jnp.stack([jnp.tile(s1e, b_blk), jnp.tile(b1e, b_blk),
                      jnp.tile(s2e, b_blk), jnp.tile(b2e, b_blk)], axis=1)
    sb3 = jnp.stack([jnp.tile(s3e, b_blk), jnp.tile(b3e, b_blk)], axis=1)

    x2d = x.reshape(N * Cin, S)
    mx = b_blk * Cin
    mh = b_blk * P
    grid = (N // b_blk,)

    ops = (w1bd, w2bd, w3bd, sb12, sb3)
    weight_specs = [pl.BlockSpec(a.shape, lambda g, nd=a.ndim: (0,) * nd)
                    for a in ops]
    in_specs = [pl.BlockSpec((mx, S), lambda g: (g, 0))] + weight_specs
    out_specs = pl.BlockSpec((mx, S), lambda g: (g, 0))

    body = functools.partial(_bottleneck_body, d_size=D, h_size=H, w_size=W,
                             b_blk=b_blk, planes=P)
    y2d = pl.pallas_call(
        body,
        out_shape=jax.ShapeDtypeStruct((N * Cin, S), x.dtype),
        grid_spec=pltpu.PrefetchScalarGridSpec(
            num_scalar_prefetch=0,
            grid=grid,
            in_specs=in_specs,
            out_specs=out_specs,
            scratch_shapes=[
                pltpu.VMEM((mh, S + 256), cdt),
                pltpu.VMEM((9 * mh, S), cdt),
            ]),
        compiler_params=pltpu.CompilerParams(
            dimension_semantics=("parallel",),
            vmem_limit_bytes=64 << 20),
    )(x2d, *ops)

    return y2d.reshape(N, Cin, D, H, W)


# speedup vs baseline: 4.0356x; 3.0730x over previous
import functools

import numpy as np
import jax
import jax.numpy as jnp
from jax.experimental import pallas as pl
from jax.experimental.pallas import tpu as pltpu


def _bottleneck_body(x_ref, w1_ref, w2_ref, w3_ref, sb12_ref, sb3_ref,
                     o_ref, hpad_ref, r2_ref, *, d_size, h_size, w_size,
                     b_blk, planes):
    s_size = x_ref.shape[2]
    mh = hpad_ref.shape[0]
    cdt = r2_ref.dtype

    x = x_ref[...].reshape(-1, s_size)

    h1 = jnp.dot(w1_ref[...], x.astype(cdt),
                 preferred_element_type=jnp.float32)
    h1 = jnp.maximum(h1 * sb12_ref[:, 0:1] + sb12_ref[:, 1:2], 0.0)

    hpad_ref[:, 0:128] = jnp.zeros((mh, 128), cdt)
    hpad_ref[:, s_size + 128:s_size + 256] = jnp.zeros((mh, 128), cdt)
    hpad_ref[:, 128:s_size + 128] = h1.astype(cdt)

    lane = jax.lax.broadcasted_iota(jnp.int32, (1, s_size), 1)
    h_of_lane = (lane // w_size) % h_size
    w_of_lane = lane % w_size

    for kd in range(3):
        for kh in range(3):
            t = kd * 3 + kh
            off = 128 + (kd - 1) * h_size * w_size + (kh - 1) * w_size
            src = hpad_ref[:, off:off + s_size]
            if kh == 0:
                src = jnp.where(h_of_lane != 0, src, 0)
            elif kh == 2:
                src = jnp.where(h_of_lane != h_size - 1, src, 0)
            r2_ref[t * mh:(t + 1) * mh, :] = src

    r2 = r2_ref[...]
    y0 = jnp.dot(w2_ref[0], r2, preferred_element_type=jnp.float32)
    y1 = jnp.dot(w2_ref[1], r2, preferred_element_type=jnp.float32)
    y2 = jnp.dot(w2_ref[2], r2, preferred_element_type=jnp.float32)

    h2 = y1
    h2 = h2 + jnp.where(w_of_lane != 0, jnp.roll(y0, 1, axis=1), 0.0)
    h2 = h2 + jnp.where(w_of_lane != w_size - 1, jnp.roll(y2, -1, axis=1), 0.0)
    h2 = jnp.maximum(h2 * sb12_ref[:, 2:3] + sb12_ref[:, 3:4], 0.0)

    h3 = jnp.dot(w3_ref[...], h2.astype(cdt),
                 preferred_element_type=jnp.float32)
    h3 = h3 * sb3_ref[:, 0:1] + sb3_ref[:, 1:2]
    o_ref[...] = jnp.maximum(h3 + x, 0.0).astype(
        o_ref.dtype).reshape(o_ref.shape)


def kernel(x, w1p, s1p, b1p, w2f, s2t, b2t, w3b, s3t, b3t):
    N, Cin, D, H, W = x.shape
    S = D * H * W
    P = w2f.shape[1] // (H * W)
    Wp = W + 2
    rowp = w1p.shape[1]
    cdt = w1p.dtype

    base = (Wp + 1) * P
    w1e = w1p[:Cin, base:base + P]
    s1e = s1p[0, base:base + P]
    b1e = b1p[0, base:base + P]

    taps = np.array([kh * Wp + kw for kh in range(3) for kw in range(3)])
    w2r = w2f[:, :P].reshape(3, rowp // P, P, P)
    w2small = w2r[:, taps].reshape(3, 3, 3, P, P)
    s2e = s2t[0, :P]
    b2e = b2t[0, :P]

    w3e = w3b[:P, :Cin]
    s3e = s3t[0, :Cin]
    b3e = b3t[0, :Cin]

    b_blk = 16
    while N % b_blk:
        b_blk //= 2

    def _bd_mask(br, bc, per):
        i = np.arange(b_blk * br)[:, None] // br
        j = np.arange(per * b_blk * bc)[None, :] % (b_blk * bc) // bc
        return (i == j).astype(np.float32)

    w1bd = (jnp.tile(w1e.T, (b_blk, b_blk))
            * _bd_mask(P, Cin, 1)).astype(cdt)
    w3bd = (jnp.tile(w3e.T, (b_blk, b_blk))
            * _bd_mask(Cin, P, 1)).astype(cdt)
    w2c = jnp.transpose(w2small, (2, 4, 0, 1, 3)).reshape(3, P, 9, P)
    w2row = jnp.broadcast_to(w2c[:, :, :, None, :],
                             (3, P, 9, b_blk, P)).reshape(3, P, 9 * b_blk * P)
    w2bd = (jnp.tile(w2row, (1, b_blk, 1))
            * _bd_mask(P, P, 9)[None]).astype(cdt)

    sb12 = jnp.stack([jnp.tile(s1e, b_blk), jnp.tile(b1e, b_blk),
                      jnp.tile(s2e, b_blk), jnp.tile(b2e, b_blk)], axis=1)
    sb3 = jnp.stack([jnp.tile(s3e, b_blk), jnp.tile(b3e, b_blk)], axis=1)

    x3d = x.reshape(N, Cin, S)
    mx = b_blk * Cin
    mh = b_blk * P
    grid = (N // b_blk,)

    ops = (w1bd, w2bd, w3bd, sb12, sb3)
    weight_specs = [pl.BlockSpec(a.shape, lambda g, nd=a.ndim: (0,) * nd)
                    for a in ops]
    in_specs = [pl.BlockSpec((b_blk, Cin, S), lambda g: (g, 0, 0))] + weight_specs
    out_specs = pl.BlockSpec((b_blk, Cin, S), lambda g: (g, 0, 0))

    body = functools.partial(_bottleneck_body, d_size=D, h_size=H, w_size=W,
                             b_blk=b_blk, planes=P)
    y3d = pl.pallas_call(
        body,
        out_shape=jax.ShapeDtypeStruct((N, Cin, S), x.dtype),
        grid_spec=pltpu.PrefetchScalarGridSpec(
            num_scalar_prefetch=0,
            grid=grid,
            in_specs=in_specs,
            out_specs=out_specs,
            scratch_shapes=[
                pltpu.VMEM((mh, S + 256), cdt),
                pltpu.VMEM((9 * mh, S), cdt),
            ]),
        compiler_params=pltpu.CompilerParams(
            dimension_semantics=("parallel",),
            vmem_limit_bytes=64 << 20),
    )(x3d, *ops)

    return y3d.reshape(N, Cin, D, H, W)
